# initial kernel scaffold (unmeasured)
import jax
import jax.numpy as jnp
from jax import lax
from jax.experimental import pallas as pl
from jax.experimental.pallas import tpu as pltpu

SEQ = 2048
SEQ_HALF = 1024
K = 4096
N = 8192
N_CHUNKS = 8
CH = N // N_CHUNKS


def kernel(O, Wo):
    o2 = O.reshape(SEQ, K).astype(jnp.bfloat16)
    w2 = Wo.astype(jnp.bfloat16)

    def body(o_ref, w_ref, out_ref, send_buf, recv_buf, send_sems, recv_sems):
        my_x = lax.axis_index("x")
        my_y = lax.axis_index("y")
        my_z = lax.axis_index("z")
        other_z = 1 - my_z
        partner = (my_x, my_y, other_z)

        barrier_sem = pltpu.get_barrier_semaphore()
        pl.semaphore_signal(
            barrier_sem, inc=1, device_id=partner,
            device_id_type=pl.DeviceIdType.MESH,
        )
        pl.semaphore_wait(barrier_sem, 1)

        o_other = o_ref[pl.ds(other_z * SEQ_HALF, SEQ_HALF), :]
        o_mine = o_ref[pl.ds(my_z * SEQ_HALF, SEQ_HALF), :]

        rdmas = []
        for j in range(N_CHUNKS):
            send_buf[j] = jnp.dot(
                o_other, w_ref[:, j * CH:(j + 1) * CH],
                preferred_element_type=jnp.float32,
            ).astype(jnp.bfloat16)
            rdma = pltpu.make_async_remote_copy(
                src_ref=send_buf.at[j],
                dst_ref=recv_buf.at[j],
                send_sem=send_sems.at[j],
                recv_sem=recv_sems.at[j],
                device_id=partner,
                device_id_type=pl.DeviceIdType.MESH,
            )
            rdma.start()
            rdmas.append(rdma)

        for j in range(N_CHUNKS):
            out_ref[:, j * CH:(j + 1) * CH] = jnp.dot(
                o_mine, w_ref[:, j * CH:(j + 1) * CH],
                preferred_element_type=jnp.float32,
            )

        for j in range(N_CHUNKS):
            rdmas[j].wait_recv()
            sl = pl.ds(j * CH, CH)
            out_ref[:, sl] = out_ref[:, sl] + recv_buf[j].astype(jnp.float32)

        for j in range(N_CHUNKS):
            rdmas[j].wait_send()

    out = pl.pallas_call(
        body,
        out_shape=jax.ShapeDtypeStruct((SEQ_HALF, N), jnp.float32),
        in_specs=[
            pl.BlockSpec(memory_space=pltpu.VMEM),
            pl.BlockSpec(memory_space=pltpu.VMEM),
        ],
        out_specs=pl.BlockSpec(memory_space=pltpu.VMEM),
        scratch_shapes=[
            pltpu.VMEM((N_CHUNKS, SEQ_HALF, CH), jnp.bfloat16),
            pltpu.VMEM((N_CHUNKS, SEQ_HALF, CH), jnp.bfloat16),
            pltpu.SemaphoreType.DMA((N_CHUNKS,)),
            pltpu.SemaphoreType.DMA((N_CHUNKS,)),
        ],
        compiler_params=pltpu.CompilerParams(collective_id=0),
    )(o2, w2)
    return out.reshape(1, SEQ_HALF, N)


# baseline (device time: 414553 ns/iter reference)
import jax
import jax.numpy as jnp
from jax import lax
from jax.experimental import pallas as pl
from jax.experimental.pallas import tpu as pltpu

SEQ = 2048
SEQ_HALF = 1024
K = 4096
N = 8192
N_CHUNKS = 16
CH = N // N_CHUNKS
ROWT = 512


def kernel(O, Wo):
    o2 = O.reshape(SEQ, K).astype(jnp.bfloat16)
    w2 = Wo.astype(jnp.bfloat16)

    def body(o_ref, w_ref, out_ref, send_buf, recv_buf, send_sems, recv_sems):
        j = pl.program_id(0)
        my_x = lax.axis_index("x")
        my_y = lax.axis_index("y")
        my_z = lax.axis_index("z")
        other_z = 1 - my_z
        partner = (my_x, my_y, other_z)
        slot = lax.rem(j, 2)

        @pl.when(j == 0)
        def _():
            barrier_sem = pltpu.get_barrier_semaphore()
            pl.semaphore_signal(
                barrier_sem, inc=1, device_id=partner,
                device_id_type=pl.DeviceIdType.MESH,
            )
            pl.semaphore_wait(barrier_sem, 1)

        @pl.when(j >= 2)
        def _():
            pltpu.make_async_remote_copy(
                src_ref=send_buf.at[slot],
                dst_ref=recv_buf.at[slot],
                send_sem=send_sems.at[slot],
                recv_sem=recv_sems.at[slot],
                device_id=partner,
                device_id_type=pl.DeviceIdType.MESH,
            ).wait_send()

        for t in range(SEQ_HALF // ROWT):
            send_buf[slot, pl.ds(t * ROWT, ROWT), :] = jnp.dot(
                o_ref[pl.ds(other_z * SEQ_HALF + t * ROWT, ROWT), :],
                w_ref[:, :],
                preferred_element_type=jnp.float32,
            ).astype(jnp.bfloat16)
        rdma = pltpu.make_async_remote_copy(
            src_ref=send_buf.at[slot],
            dst_ref=recv_buf.at[slot],
            send_sem=send_sems.at[slot],
            recv_sem=recv_sems.at[slot],
            device_id=partner,
            device_id_type=pl.DeviceIdType.MESH,
        )
        rdma.start()

        for t in range(SEQ_HALF // ROWT):
            out_ref[pl.ds(t * ROWT, ROWT), :] = jnp.dot(
                o_ref[pl.ds(my_z * SEQ_HALF + t * ROWT, ROWT), :],
                w_ref[:, :],
                preferred_element_type=jnp.float32,
            )

        rdma.wait_recv()
        out_ref[:, :] = out_ref[:, :] + recv_buf[slot].astype(jnp.float32)

        @pl.when(j == N_CHUNKS - 1)
        def _():
            for s in (slot, 1 - slot):
                pltpu.make_async_remote_copy(
                    src_ref=send_buf.at[s],
                    dst_ref=recv_buf.at[s],
                    send_sem=send_sems.at[s],
                    recv_sem=recv_sems.at[s],
                    device_id=partner,
                    device_id_type=pl.DeviceIdType.MESH,
                ).wait_send()

    out = pl.pallas_call(
        body,
        grid=(N_CHUNKS,),
        out_shape=jax.ShapeDtypeStruct((SEQ_HALF, N), jnp.float32),
        in_specs=[
            pl.BlockSpec((SEQ, K), lambda j: (0, 0)),
            pl.BlockSpec((K, CH), lambda j: (0, j)),
        ],
        out_specs=pl.BlockSpec((SEQ_HALF, CH), lambda j: (0, j)),
        scratch_shapes=[
            pltpu.VMEM((2, SEQ_HALF, CH), jnp.bfloat16),
            pltpu.VMEM((2, SEQ_HALF, CH), jnp.bfloat16),
            pltpu.SemaphoreType.DMA((2,)),
            pltpu.SemaphoreType.DMA((2,)),
        ],
        compiler_params=pltpu.CompilerParams(
            collective_id=0,
            dimension_semantics=("arbitrary",),
            vmem_limit_bytes=36 * 1024 * 1024,
        ),
    )(o2, w2)
    return out.reshape(1, SEQ_HALF, N)


# device time: 310537 ns/iter; 1.3350x vs baseline; 1.3350x over previous
import jax
import jax.numpy as jnp
from jax import lax
from jax.experimental import pallas as pl
from jax.experimental.pallas import tpu as pltpu

SEQ = 2048
SEQ_HALF = 1024
K = 4096
N = 8192
N_CHUNKS = 16
CH = N // N_CHUNKS
ROWT = 512
N_SEND_SLOTS = 2
N_RECV_SLOTS = 4


def kernel(O, Wo):
    o2 = O.reshape(SEQ, K).astype(jnp.bfloat16)
    w2 = Wo.astype(jnp.bfloat16)

    def body(o_ref, w_ref, out_ref, send_buf, recv_buf, stage,
             send_sems, recv_sems):
        j = pl.program_id(0)
        my_x = lax.axis_index("x")
        my_y = lax.axis_index("y")
        my_z = lax.axis_index("z")
        other_z = 1 - my_z
        partner = (my_x, my_y, other_z)
        s_slot = lax.rem(j, N_SEND_SLOTS)
        r_slot = lax.rem(j, N_RECV_SLOTS)

        def mk(s, r):
            return pltpu.make_async_remote_copy(
                src_ref=send_buf.at[s],
                dst_ref=recv_buf.at[r],
                send_sem=send_sems.at[s],
                recv_sem=recv_sems.at[r],
                device_id=partner,
                device_id_type=pl.DeviceIdType.MESH,
            )

        @pl.when(j == 0)
        def _():
            barrier_sem = pltpu.get_barrier_semaphore()
            pl.semaphore_signal(
                barrier_sem, inc=1, device_id=partner,
                device_id_type=pl.DeviceIdType.MESH,
            )
            pl.semaphore_wait(barrier_sem, 1)

        @pl.when(j < N_CHUNKS)
        def _():
            @pl.when(j >= N_SEND_SLOTS)
            def _():
                mk(s_slot, r_slot).wait_send()

            for t in range(SEQ_HALF // ROWT):
                send_buf[s_slot, pl.ds(t * ROWT, ROWT), :] = jnp.dot(
                    o_ref[pl.ds(other_z * SEQ_HALF + t * ROWT, ROWT), :],
                    w_ref[:, :],
                    preferred_element_type=jnp.float32,
                ).astype(jnp.bfloat16)
            mk(s_slot, r_slot).start()

            for t in range(SEQ_HALF // ROWT):
                stage[s_slot, pl.ds(t * ROWT, ROWT), :] = jnp.dot(
                    o_ref[pl.ds(my_z * SEQ_HALF + t * ROWT, ROWT), :],
                    w_ref[:, :],
                    preferred_element_type=jnp.float32,
                )

        @pl.when(j > 0)
        def _():
            ps = lax.rem(j - 1, N_SEND_SLOTS)
            pr = lax.rem(j - 1, N_RECV_SLOTS)
            mk(ps, pr).wait_recv()
            out_ref[:, :] = stage[ps] + recv_buf[pr].astype(jnp.float32)

        @pl.when(j == N_CHUNKS)
        def _():
            for s in range(N_SEND_SLOTS):
                mk(s, 0).wait_send()

    out = pl.pallas_call(
        body,
        grid=(N_CHUNKS + 1,),
        out_shape=jax.ShapeDtypeStruct((SEQ_HALF, N), jnp.float32),
        in_specs=[
            pl.BlockSpec((SEQ, K), lambda j: (0, 0)),
            pl.BlockSpec((K, CH), lambda j: (0, jnp.minimum(j, N_CHUNKS - 1))),
        ],
        out_specs=pl.BlockSpec(
            (SEQ_HALF, CH), lambda j: (0, jnp.maximum(j - 1, 0))
        ),
        scratch_shapes=[
            pltpu.VMEM((N_SEND_SLOTS, SEQ_HALF, CH), jnp.bfloat16),
            pltpu.VMEM((N_RECV_SLOTS, SEQ_HALF, CH), jnp.bfloat16),
            pltpu.VMEM((N_SEND_SLOTS, SEQ_HALF, CH), jnp.float32),
            pltpu.SemaphoreType.DMA((N_SEND_SLOTS,)),
            pltpu.SemaphoreType.DMA((N_RECV_SLOTS,)),
        ],
        compiler_params=pltpu.CompilerParams(
            collective_id=0,
            dimension_semantics=("arbitrary",),
            vmem_limit_bytes=35 * 1024 * 1024,
        ),
    )(o2, w2)
    return out.reshape(1, SEQ_HALF, N)


# device time: 243354 ns/iter; 1.7035x vs baseline; 1.2761x over previous
import jax
import jax.numpy as jnp
from jax import lax
from jax.experimental import pallas as pl
from jax.experimental.pallas import tpu as pltpu

SEQ = 2048
SEQ_HALF = 1024
K = 4096
N = 8192
N_CHUNKS = 32
CH = N // N_CHUNKS
ROWT = 256
N_SEND_SLOTS = 2
N_RECV_SLOTS = 4


def kernel(O, Wo):
    o2 = O.reshape(SEQ, K).astype(jnp.bfloat16)

    def body(o_ref, w_ref, out_ref, send_buf, recv_buf, stage, wbf,
             send_sems, recv_sems):
        j = pl.program_id(0)
        my_x = lax.axis_index("x")
        my_y = lax.axis_index("y")
        my_z = lax.axis_index("z")
        other_z = 1 - my_z
        partner = (my_x, my_y, other_z)
        s_slot = lax.rem(j, N_SEND_SLOTS)
        r_slot = lax.rem(j, N_RECV_SLOTS)

        def mk(s, r):
            return pltpu.make_async_remote_copy(
                src_ref=send_buf.at[s],
                dst_ref=recv_buf.at[r],
                send_sem=send_sems.at[s],
                recv_sem=recv_sems.at[r],
                device_id=partner,
                device_id_type=pl.DeviceIdType.MESH,
            )

        @pl.when(j == 0)
        def _():
            barrier_sem = pltpu.get_barrier_semaphore()
            pl.semaphore_signal(
                barrier_sem, inc=1, device_id=partner,
                device_id_type=pl.DeviceIdType.MESH,
            )
            pl.semaphore_wait(barrier_sem, 1)

        @pl.when(j < N_CHUNKS)
        def _():
            for c in range(2):
                sl = pl.ds(c * (CH // 2), CH // 2)
                wbf[:, sl] = w_ref[:, sl].astype(jnp.bfloat16)

            @pl.when(j >= N_SEND_SLOTS)
            def _():
                mk(s_slot, r_slot).wait_send()

            for t in range(SEQ_HALF // ROWT):
                send_buf[s_slot, pl.ds(t * ROWT, ROWT), :] = jnp.dot(
                    o_ref[pl.ds(other_z * SEQ_HALF + t * ROWT, ROWT), :],
                    wbf[:, :],
                    preferred_element_type=jnp.float32,
                ).astype(jnp.bfloat16)
            mk(s_slot, r_slot).start()

            for t in range(SEQ_HALF // ROWT):
                stage[s_slot, pl.ds(t * ROWT, ROWT), :] = jnp.dot(
                    o_ref[pl.ds(my_z * SEQ_HALF + t * ROWT, ROWT), :],
                    wbf[:, :],
                    preferred_element_type=jnp.float32,
                ).astype(jnp.bfloat16)

        @pl.when(j > 0)
        def _():
            ps = lax.rem(j - 1, N_SEND_SLOTS)
            pr = lax.rem(j - 1, N_RECV_SLOTS)
            mk(ps, pr).wait_recv()
            for t in range(SEQ_HALF // ROWT):
                sl = pl.ds(t * ROWT, ROWT)
                out_ref[sl, :] = (
                    stage[ps, sl, :].astype(jnp.float32)
                    + recv_buf[pr, sl, :].astype(jnp.float32)
                )

        @pl.when(j == N_CHUNKS)
        def _():
            for s in range(N_SEND_SLOTS):
                mk(s, 0).wait_send()

    out = pl.pallas_call(
        body,
        grid=(N_CHUNKS + 1,),
        out_shape=jax.ShapeDtypeStruct((SEQ_HALF, N), jnp.float32),
        in_specs=[
            pl.BlockSpec((SEQ, K), lambda j: (0, 0)),
            pl.BlockSpec((K, CH), lambda j: (0, jnp.minimum(j, N_CHUNKS - 1))),
        ],
        out_specs=pl.BlockSpec(
            (SEQ_HALF, CH), lambda j: (0, jnp.maximum(j - 1, 0))
        ),
        scratch_shapes=[
            pltpu.VMEM((N_SEND_SLOTS, SEQ_HALF, CH), jnp.bfloat16),
            pltpu.VMEM((N_RECV_SLOTS, SEQ_HALF, CH), jnp.bfloat16),
            pltpu.VMEM((N_SEND_SLOTS, SEQ_HALF, CH), jnp.bfloat16),
            pltpu.VMEM((K, CH), jnp.bfloat16),
            pltpu.SemaphoreType.DMA((N_SEND_SLOTS,)),
            pltpu.SemaphoreType.DMA((N_RECV_SLOTS,)),
        ],
        compiler_params=pltpu.CompilerParams(
            collective_id=0,
            dimension_semantics=("arbitrary",),
            vmem_limit_bytes=37 * 1024 * 1024,
        ),
    )(o2, Wo)
    return out.reshape(1, SEQ_HALF, N)


# device time: 243182 ns/iter; 1.7047x vs baseline; 1.0007x over previous
import jax
import jax.numpy as jnp
from jax import lax
from jax.experimental import pallas as pl
from jax.experimental.pallas import tpu as pltpu

SEQ = 2048
SEQ_HALF = 1024
K = 4096
N = 8192
N_CHUNKS = 32
CH = N // N_CHUNKS
ROWT = 512
N_SEND_SLOTS = 2
N_RECV_SLOTS = 4


def kernel(O, Wo):
    o2 = O.reshape(SEQ, K).astype(jnp.bfloat16)

    def body(o_ref, w_ref, out_ref, send_buf, recv_buf, stage, wbf,
             send_sems, recv_sems):
        j = pl.program_id(0)
        my_x = lax.axis_index("x")
        my_y = lax.axis_index("y")
        my_z = lax.axis_index("z")
        other_z = 1 - my_z
        partner = (my_x, my_y, other_z)
        s_slot = lax.rem(j, N_SEND_SLOTS)
        r_slot = lax.rem(j, N_RECV_SLOTS)

        def mk(s, r):
            return pltpu.make_async_remote_copy(
                src_ref=send_buf.at[s],
                dst_ref=recv_buf.at[r],
                send_sem=send_sems.at[s],
                recv_sem=recv_sems.at[r],
                device_id=partner,
                device_id_type=pl.DeviceIdType.MESH,
            )

        @pl.when(j == 0)
        def _():
            barrier_sem = pltpu.get_barrier_semaphore()
            pl.semaphore_signal(
                barrier_sem, inc=1, device_id=partner,
                device_id_type=pl.DeviceIdType.MESH,
            )
            pl.semaphore_wait(barrier_sem, 1)

        @pl.when(j < N_CHUNKS)
        def _():
            for c in range(2):
                sl = pl.ds(c * (CH // 2), CH // 2)
                wbf[:, sl] = w_ref[:, sl].astype(jnp.bfloat16)

            @pl.when(j >= N_SEND_SLOTS)
            def _():
                mk(s_slot, r_slot).wait_send()

            for t in range(SEQ_HALF // ROWT):
                send_buf[s_slot, pl.ds(t * ROWT, ROWT), :] = jnp.dot(
                    o_ref[pl.ds(other_z * SEQ_HALF + t * ROWT, ROWT), :],
                    wbf[:, :],
                    preferred_element_type=jnp.float32,
                ).astype(jnp.bfloat16)
            mk(s_slot, r_slot).start()

            for t in range(SEQ_HALF // ROWT):
                stage[s_slot, pl.ds(t * ROWT, ROWT), :] = jnp.dot(
                    o_ref[pl.ds(my_z * SEQ_HALF + t * ROWT, ROWT), :],
                    wbf[:, :],
                    preferred_element_type=jnp.float32,
                ).astype(jnp.bfloat16)

        @pl.when(j > 0)
        def _():
            ps = lax.rem(j - 1, N_SEND_SLOTS)
            pr = lax.rem(j - 1, N_RECV_SLOTS)
            mk(ps, pr).wait_recv()
            for t in range(SEQ_HALF // ROWT):
                sl = pl.ds(t * ROWT, ROWT)
                out_ref[sl, :] = (
                    stage[ps, sl, :].astype(jnp.float32)
                    + recv_buf[pr, sl, :].astype(jnp.float32)
                )

        @pl.when(j == N_CHUNKS)
        def _():
            for s in range(N_SEND_SLOTS):
                mk(s, 0).wait_send()

    out = pl.pallas_call(
        body,
        grid=(N_CHUNKS + 1,),
        out_shape=jax.ShapeDtypeStruct((SEQ_HALF, N), jnp.float32),
        in_specs=[
            pl.BlockSpec((SEQ, K), lambda j: (0, 0)),
            pl.BlockSpec((K, CH), lambda j: (0, jnp.minimum(j, N_CHUNKS - 1))),
        ],
        out_specs=pl.BlockSpec(
            (SEQ_HALF, CH), lambda j: (0, jnp.maximum(j - 1, 0))
        ),
        scratch_shapes=[
            pltpu.VMEM((N_SEND_SLOTS, SEQ_HALF, CH), jnp.bfloat16),
            pltpu.VMEM((N_RECV_SLOTS, SEQ_HALF, CH), jnp.bfloat16),
            pltpu.VMEM((N_SEND_SLOTS, SEQ_HALF, CH), jnp.bfloat16),
            pltpu.VMEM((K, CH), jnp.bfloat16),
            pltpu.SemaphoreType.DMA((N_SEND_SLOTS,)),
            pltpu.SemaphoreType.DMA((N_RECV_SLOTS,)),
        ],
        compiler_params=pltpu.CompilerParams(
            collective_id=0,
            dimension_semantics=("arbitrary",),
            vmem_limit_bytes=37 * 1024 * 1024,
        ),
    )(o2, Wo)
    return out.reshape(1, SEQ_HALF, N)
